# baseline (speedup 1.0000x reference)
import jax
import jax.numpy as jnp
from jax import lax
from jax.experimental import pallas as pl
from jax.experimental.pallas import tpu as pltpu
from jax.experimental.pallas import tpu_sc as plsc

K = 100000
D = 64
NSENT = 4096
LS = 50
MID_ROWS = NSENT * LS // 2

_info = plsc.get_sparse_core_info()
NC, NS, L = _info.num_cores, _info.num_subcores, _info.num_lanes
NW = NC * NS
S_PER_W = NSENT // NW
SENT_PER_SLOT = 4
N_SLOTS = S_PER_W // SENT_PER_SLOT
PAIR_ROWS = 2 * SENT_PER_SLOT * LS // 2


def _body(idx_hbm, table_hbm, out_hbm, idx_v, b0, b1, obuf, g0, g1, ssem):
    wid = lax.axis_index("s") * NC + lax.axis_index("c")
    sent0 = wid * S_PER_W

    pltpu.sync_copy(idx_hbm.at[pl.ds(sent0, S_PER_W)], idx_v)

    bufs = (b0, b1)
    gsems = (g0, g1)

    def sg(t, b):
        for j in range(SENT_PER_SLOT):
            pltpu.async_copy(
                table_hbm.at[idx_v.at[t * SENT_PER_SLOT + j]],
                bufs[b].at[j],
                gsems[b],
            )

    def wg(t, b):
        for j in range(SENT_PER_SLOT):
            pltpu.make_async_copy(
                table_hbm.at[idx_v.at[t * SENT_PER_SLOT + j]],
                bufs[b].at[j],
                gsems[b],
            ).wait()

    def pack(half_t, b):
        buf = bufs[b]

        @plsc.parallel_loop(0, LS // 2)
        def _rows(m):
            for j in range(SENT_PER_SLOT):
                o = half_t * (SENT_PER_SLOT * LS // 2) + j * (LS // 2)
                for h in range(2):
                    for k in range(D // L):
                        s = pl.ds(k * L, L)
                        d = pl.ds(h * D + k * L, L)
                        obuf[o + m, d] = jnp.maximum(buf[j, 2 * m + h, s],
                                                     0.0)

    def store_pair(p):
        pltpu.async_copy(
            obuf,
            out_hbm.at[pl.ds(sent0 * (LS // 2) + p * PAIR_ROWS, PAIR_ROWS)],
            ssem,
        )
        pltpu.make_async_copy(
            obuf,
            out_hbm.at[pl.ds(sent0 * (LS // 2) + p * PAIR_ROWS, PAIR_ROWS)],
            ssem,
        ).wait()

    sg(0, 0)

    def pair(k, carry):
        t = 2 * k
        sg(t + 1, 1)
        wg(t, 0)
        pack(0, 0)
        sg(t + 2, 0)
        wg(t + 1, 1)
        pack(1, 1)
        store_pair(k)
        return carry

    lax.fori_loop(0, N_SLOTS // 2 - 1, pair, 0)

    t = N_SLOTS - 2
    sg(t + 1, 1)
    wg(t, 0)
    pack(0, 0)
    wg(t + 1, 1)
    pack(1, 1)
    store_pair(N_SLOTS // 2 - 1)


@jax.jit
def _run(tags, table128):
    mesh = plsc.VectorSubcoreMesh(core_axis_name="c", subcore_axis_name="s")
    return pl.kernel(
        _body,
        out_type=jax.ShapeDtypeStruct((MID_ROWS, 2 * D), jnp.float32),
        mesh=mesh,
        scratch_types=[
            pltpu.VMEM((S_PER_W, LS), jnp.int32),
            pltpu.VMEM((SENT_PER_SLOT, LS, 2 * D), jnp.float32),
            pltpu.VMEM((SENT_PER_SLOT, LS, 2 * D), jnp.float32),
            pltpu.VMEM((PAIR_ROWS, 2 * D), jnp.float32),
            pltpu.SemaphoreType.DMA,
            pltpu.SemaphoreType.DMA,
            pltpu.SemaphoreType.DMA,
        ],
        compiler_params=pltpu.CompilerParams(use_tc_tiling_on_sc=True),
    )(tags, table128)


def kernel(preprocessed_tags, embedding_weight):
    tags = preprocessed_tags.astype(jnp.int32)
    table128 = jnp.pad(embedding_weight, ((0, 0), (0, D)))
    mid = _run(tags, table128)
    return mid.reshape(NSENT, LS, D)


# speedup vs baseline: 1.0887x; 1.0887x over previous
import jax
import jax.numpy as jnp
from jax import lax
from jax.experimental import pallas as pl
from jax.experimental.pallas import tpu as pltpu
from jax.experimental.pallas import tpu_sc as plsc

K = 100000
D = 64
NSENT = 4096
LS = 50
B = NSENT * LS

_info = plsc.get_sparse_core_info()
NC, NS, L = _info.num_cores, _info.num_subcores, _info.num_lanes
NW = NC * NS
S_PER_W = NSENT // NW
SENT_PER_SLOT = 8
ROWS = SENT_PER_SLOT * LS
N_SLOTS = S_PER_W // SENT_PER_SLOT
NBUF = 4
AHEAD = 2


def _body(idx_hbm, table_hbm, out_hbm, idx_v, b0, b1, b2, b3,
          g0, g1, g2, g3, s0, s1, s2, s3):
    wid = lax.axis_index("s") * NC + lax.axis_index("c")
    sent0 = wid * S_PER_W

    pltpu.sync_copy(idx_hbm.at[pl.ds(sent0, S_PER_W)], idx_v)

    bufs = (b0, b1, b2, b3)
    gsems = (g0, g1, g2, g3)
    ssems = (s0, s1, s2, s3)

    def sg(t, b):
        for j in range(SENT_PER_SLOT):
            pltpu.async_copy(
                table_hbm.at[idx_v.at[t * SENT_PER_SLOT + j]],
                bufs[b].at[j],
                gsems[b],
            )

    def wg(b):
        for j in range(SENT_PER_SLOT):
            pltpu.make_async_copy(
                table_hbm.at[pl.ds(0, LS)],
                bufs[b].at[j],
                gsems[b],
            ).wait()

    def ss(t, b):
        pltpu.async_copy(
            bufs[b],
            out_hbm.at[pl.ds(sent0 + t * SENT_PER_SLOT, SENT_PER_SLOT)],
            ssems[b],
        )

    def ws(b):
        pltpu.make_async_copy(
            bufs[b],
            out_hbm.at[pl.ds(sent0, SENT_PER_SLOT)],
            ssems[b],
        ).wait()

    def relu(b):
        buf = bufs[b]

        @plsc.parallel_loop(0, LS, step=2)
        def _relu_rows(i):
            for j in range(SENT_PER_SLOT):
                for r in range(2):
                    for k in range(D // L):
                        s = pl.ds(k * L, L)
                        buf[j, i + r, s] = jnp.maximum(buf[j, i + r, s], 0.0)

    sg(0, 0)
    sg(1, 1)
    sg(2, 2)
    wg(0)
    relu(0)
    ss(0, 0)
    sg(3, 3)
    wg(1)
    relu(1)
    ss(1, 1)

    def outer(k, carry):
        for j in range(4):
            t = 2 + k * 4 + j
            b = (2 + j) % NBUF
            nb = (b + AHEAD) % NBUF
            ws(nb)
            sg(t + AHEAD, nb)
            wg(b)
            relu(b)
            ss(t, b)
        return carry

    lax.fori_loop(0, (N_SLOTS - 4) // 4, outer, 0)

    for t in (N_SLOTS - 2, N_SLOTS - 1):
        b = t % NBUF
        ws((b + 2) % NBUF)
        wg(b)
        relu(b)
        ss(t, b)
    ws((N_SLOTS - 2) % NBUF)
    ws((N_SLOTS - 1) % NBUF)


@jax.jit
def _run(tags, table):
    mesh = plsc.VectorSubcoreMesh(core_axis_name="c", subcore_axis_name="s")
    return pl.kernel(
        _body,
        out_type=jax.ShapeDtypeStruct((NSENT, LS, D), jnp.float32),
        mesh=mesh,
        scratch_types=[
            pltpu.VMEM((S_PER_W, LS), jnp.int32),
            pltpu.VMEM((SENT_PER_SLOT, LS, D), jnp.float32),
            pltpu.VMEM((SENT_PER_SLOT, LS, D), jnp.float32),
            pltpu.VMEM((SENT_PER_SLOT, LS, D), jnp.float32),
            pltpu.VMEM((SENT_PER_SLOT, LS, D), jnp.float32),
            pltpu.SemaphoreType.DMA,
            pltpu.SemaphoreType.DMA,
            pltpu.SemaphoreType.DMA,
            pltpu.SemaphoreType.DMA,
            pltpu.SemaphoreType.DMA,
            pltpu.SemaphoreType.DMA,
            pltpu.SemaphoreType.DMA,
            pltpu.SemaphoreType.DMA,
        ],
        compiler_params=pltpu.CompilerParams(use_tc_tiling_on_sc=False),
    )(tags, table)


def kernel(preprocessed_tags, embedding_weight):
    tags = preprocessed_tags.astype(jnp.int32)
    return _run(tags, embedding_weight)
